# baseline (speedup 1.0000x reference)
import functools

import jax
import jax.numpy as jnp
from jax import lax
from jax.experimental import pallas as pl
from jax.experimental.pallas import tpu as pltpu
from jax.experimental.pallas import tpu_sc as plsc

B, C, D = 1024, 512, 64
BT = 256
BS = B - BT
BBLK = 32
NC, NS, L = 2, 16, 16
NW = NC * NS
BPW = BT // NW
BPW1 = BS // NW
NTILE = B * C // 8
TPR = C // 8


def _cls_body(x_ref, cls_ref):
    xb = x_ref[...]
    s = jnp.sum(xb * xb, axis=2)
    cls_ref[...] = jnp.sqrt(s).reshape(BBLK * 4, 128)


def _classes(x):
    return pl.pallas_call(
        _cls_body,
        grid=(BT // BBLK,),
        in_specs=[pl.BlockSpec((BBLK, C, D), lambda i: (i, 0, 0))],
        out_specs=pl.BlockSpec((BBLK * 4, 128), lambda i: (i, 0)),
        out_shape=jax.ShapeDtypeStruct((BT * C // 128, 128), jnp.float32),
    )(x)


def _sqrt_body(a_ref, o_ref):
    o_ref[...] = jnp.sqrt(a_ref[...])


def _sqrt_tc(a):
    n = a.shape[0] // 128
    return pl.pallas_call(
        _sqrt_body,
        grid=(4,),
        in_specs=[pl.BlockSpec((n // 4, 128), lambda i: (i, 0))],
        out_specs=pl.BlockSpec((n // 4, 128), lambda i: (i, 0)),
        out_shape=jax.ShapeDtypeStruct((n, 128), jnp.float32),
    )(a.reshape(n, 128)).reshape(a.shape)


_mesh = plsc.VectorSubcoreMesh(core_axis_name="c", subcore_axis_name="s")


@functools.partial(
    pl.kernel,
    out_type=[
        jax.ShapeDtypeStruct((BS * C,), jnp.float32),
        jax.ShapeDtypeStruct((BS,), jnp.int32),
        jax.ShapeDtypeStruct((BS * D,), jnp.float32),
    ],
    mesh=_mesh,
    compiler_params=pltpu.CompilerParams(needs_layout_passes=False),
    scratch_types=[
        pltpu.VMEM((TPR, 8, D), jnp.float32),
        pltpu.VMEM((C,), jnp.float32),
        pltpu.VMEM((2 * L,), jnp.int32),
        pltpu.VMEM((BPW1 * D,), jnp.float32),
        pltpu.SemaphoreType.DMA,
    ],
)
def _sc_norms(x46_hbm, clssq_hbm, idx_hbm, masked_hbm,
              slab_v, clsrow_v, idx_v, out_v, sem):
    wid = lax.axis_index("s") * NC + lax.axis_index("c")
    base = wid * BPW1
    lane = lax.broadcasted_iota(jnp.int32, (L,), 0)

    for g in range((BPW1 + L - 1) // L):
        gsize = min(L, BPW1 - g * L)

        def row_body(rr, acc):
            r = g * L + rr
            gr = BT + base + r
            pltpu.sync_copy(x46_hbm.at[pl.ds(gr * TPR, TPR)], slab_v)

            def grp(t2, carry):
                best_s, best_c = carry
                a16 = jnp.zeros((L,), jnp.float32)
                for u in range(L):
                    tidx = 2 * t2 + (u // 8)
                    v0 = slab_v[tidx, u % 8, pl.ds(0, L)]
                    v1 = slab_v[tidx, u % 8, pl.ds(L, L)]
                    v2 = slab_v[tidx, u % 8, pl.ds(2 * L, L)]
                    v3 = slab_v[tidx, u % 8, pl.ds(3 * L, L)]
                    s = jnp.sum(v0 * v0 + v1 * v1 + v2 * v2 + v3 * v3)
                    a16 = jnp.where(lane == u, s, a16)
                    cc = t2 * L + u
                    take = s > best_s
                    best_s = jnp.where(take, s, best_s)
                    best_c = jnp.where(take, cc, best_c)
                clsrow_v[pl.ds(t2 * L, L)] = a16
                return best_s, best_c

            _, cstar = lax.fori_loop(0, C // L, grp,
                                     (jnp.float32(-1.0), jnp.int32(0)))
            pltpu.sync_copy(clsrow_v,
                            clssq_hbm.at[pl.ds((base + r) * C, C)])
            for m in range(D // L):
                out_v[pl.ds(r * D + m * L, L)] = (
                    slab_v[cstar >> 3, cstar & 7, pl.ds(m * L, L)])
            return jnp.where(lane == rr, cstar, acc)

        acc = lax.fori_loop(0, gsize, row_body, jnp.zeros((L,), jnp.int32))
        idx_v[pl.ds(g * L, L)] = acc

    pltpu.sync_copy(idx_v.at[pl.ds(0, BPW1)],
                    idx_hbm.at[pl.ds(base, BPW1)])
    pltpu.sync_copy(out_v, masked_hbm.at[pl.ds(base * D, BPW1 * D)])


@functools.partial(
    pl.kernel,
    out_type=[
        jax.ShapeDtypeStruct((BT,), jnp.int32),
        jax.ShapeDtypeStruct((BT * D,), jnp.float32),
    ],
    mesh=_mesh,
    compiler_params=pltpu.CompilerParams(needs_layout_passes=False),
    scratch_types=[
        pltpu.VMEM((BPW * C,), jnp.float32),
        pltpu.VMEM((L,), jnp.int32),
        pltpu.VMEM((8, D), jnp.float32),
        pltpu.VMEM((BPW * D,), jnp.float32),
        pltpu.SemaphoreType.DMA,
    ],
)
def _sc_pick(cls_hbm, x46_hbm, idx_hbm, masked_hbm,
             cls_v, idx_v, tile_v, out_v, sem):
    wid = lax.axis_index("s") * NC + lax.axis_index("c")
    base = wid * BPW
    pltpu.sync_copy(cls_hbm.at[pl.ds(base * C, BPW * C)], cls_v)
    lane = lax.broadcasted_iota(jnp.int32, (L,), 0)

    def row_body(j, acc):
        def chunk(ci, carry):
            vmax, varg = carry
            v = cls_v[pl.ds(j * C + ci * L, L)]
            take = v > vmax
            return (jnp.where(take, v, vmax),
                    jnp.where(take, ci * L + lane, varg))

        vmax, varg = lax.fori_loop(
            0, C // L, chunk,
            (jnp.full((L,), -1.0, jnp.float32),
             jnp.zeros((L,), jnp.int32)),
        )
        m = jnp.max(vmax)
        c = jnp.min(jnp.where(vmax == m, varg, C))
        t = ((base + j) * C + c) >> 3
        pltpu.sync_copy(x46_hbm.at[t], tile_v)
        k = c & 7
        for mm in range(D // L):
            out_v[pl.ds(j * D + mm * L, L)] = tile_v[k, pl.ds(mm * L, L)]
        return jnp.where(lane == j, c, acc)

    acc = lax.fori_loop(0, BPW, row_body, jnp.zeros((L,), jnp.int32))
    idx_v[...] = acc
    pltpu.sync_copy(idx_v.at[pl.ds(0, BPW)], idx_hbm.at[pl.ds(base, BPW)])
    pltpu.sync_copy(out_v, masked_hbm.at[pl.ds(base * D, BPW * D)])


def kernel(x):
    x46 = x.reshape(NTILE, 8, D)
    clssq_sc, idx_sc, masked_sc = _sc_norms(x46)
    cls_tc = _classes(x)
    idx_tc, masked_tc = _sc_pick(cls_tc.reshape(BT * C), x46)
    cls_sc = _sqrt_tc(clssq_sc)
    cls = jnp.concatenate([cls_tc.reshape(BT, C), cls_sc.reshape(BS, C)])
    idx = jnp.concatenate([idx_tc, idx_sc])
    masked = jnp.concatenate([masked_tc.reshape(BT, 1, D),
                              masked_sc.reshape(BS, 1, D)])
    return masked, idx, cls


# speedup vs baseline: 1.0100x; 1.0100x over previous
import functools

import jax
import jax.numpy as jnp
from jax import lax
from jax.experimental import pallas as pl
from jax.experimental.pallas import tpu as pltpu
from jax.experimental.pallas import tpu_sc as plsc

B, C, D = 1024, 512, 64
NC, NS, L = 2, 16, 16
NW = NC * NS
BPW = B // NW


def _sqrt_body(a_ref, o_ref):
    o_ref[...] = jnp.sqrt(a_ref[...])


def _sqrt_tc(a):
    n = a.shape[0] // 128
    return pl.pallas_call(
        _sqrt_body,
        grid=(4,),
        in_specs=[pl.BlockSpec((n // 4, 128), lambda i: (i, 0))],
        out_specs=pl.BlockSpec((n // 4, 128), lambda i: (i, 0)),
        out_shape=jax.ShapeDtypeStruct((n, 128), jnp.float32),
    )(a.reshape(n, 128)).reshape(a.shape)


_mesh = plsc.VectorSubcoreMesh(core_axis_name="c", subcore_axis_name="s")


@functools.partial(
    pl.kernel,
    out_type=[
        jax.ShapeDtypeStruct((B * C,), jnp.float32),
        jax.ShapeDtypeStruct((B,), jnp.int32),
        jax.ShapeDtypeStruct((B * D,), jnp.float32),
    ],
    mesh=_mesh,
    compiler_params=pltpu.CompilerParams(needs_layout_passes=False),
    scratch_types=[
        pltpu.VMEM((C, D), jnp.float32),
        pltpu.VMEM((C,), jnp.float32),
        pltpu.VMEM((BPW,), jnp.int32),
        pltpu.VMEM((BPW * D,), jnp.float32),
        pltpu.SemaphoreType.DMA,
    ],
)
def _sc_norms(x_hbm, clssq_hbm, idx_hbm, masked_hbm,
              slab_v, clsrow_v, idx_v, out_v, sem):
    wid = lax.axis_index("s") * NC + lax.axis_index("c")
    base = wid * BPW
    lane = lax.broadcasted_iota(jnp.int32, (L,), 0)

    for g in range(BPW // L):

        def row_body(rr, acc):
            r = g * L + rr
            gr = base + r
            pltpu.sync_copy(x_hbm.at[gr], slab_v)

            def grp(t2, carry):
                best_s, best_c = carry
                a16 = jnp.zeros((L,), jnp.float32)
                for u in range(L):
                    cc = t2 * L + u
                    v0 = slab_v[cc, pl.ds(0, L)]
                    v1 = slab_v[cc, pl.ds(L, L)]
                    v2 = slab_v[cc, pl.ds(2 * L, L)]
                    v3 = slab_v[cc, pl.ds(3 * L, L)]
                    s = jnp.sum(v0 * v0 + v1 * v1 + v2 * v2 + v3 * v3)
                    a16 = jnp.where(lane == u, s, a16)
                    take = s > best_s
                    best_s = jnp.where(take, s, best_s)
                    best_c = jnp.where(take, cc, best_c)
                clsrow_v[pl.ds(t2 * L, L)] = a16
                return best_s, best_c

            _, cstar = lax.fori_loop(0, C // L, grp,
                                     (jnp.float32(-1.0), jnp.int32(0)))
            pltpu.sync_copy(clsrow_v, clssq_hbm.at[pl.ds(gr * C, C)])
            for m in range(D // L):
                out_v[pl.ds(r * D + m * L, L)] = slab_v[cstar, pl.ds(m * L, L)]
            return jnp.where(lane == rr, cstar, acc)

        acc = lax.fori_loop(0, L, row_body, jnp.zeros((L,), jnp.int32))
        idx_v[pl.ds(g * L, L)] = acc

    pltpu.sync_copy(idx_v, idx_hbm.at[pl.ds(base, BPW)])
    pltpu.sync_copy(out_v, masked_hbm.at[pl.ds(base * D, BPW * D)])


def kernel(x):
    clssq, idx, masked = _sc_norms(x)
    cls = _sqrt_tc(clssq)
    return masked.reshape(B, 1, D), idx, cls.reshape(B, C)


# speedup vs baseline: 1.2862x; 1.2736x over previous
import functools

import jax
import jax.numpy as jnp
from jax import lax
from jax.experimental import pallas as pl
from jax.experimental.pallas import tpu as pltpu
from jax.experimental.pallas import tpu_sc as plsc

B, C, D = 1024, 512, 64
NC, NS, L = 2, 16, 16
NW = NC * NS
BPW = B // NW
NTILE = B * C // 8
TPR = C // 8


def _sqrt_body(a_ref, o_ref):
    o_ref[...] = jnp.sqrt(a_ref[...])


def _sqrt_tc(a):
    n = a.shape[0] // 128
    return pl.pallas_call(
        _sqrt_body,
        grid=(4,),
        in_specs=[pl.BlockSpec((n // 4, 128), lambda i: (i, 0))],
        out_specs=pl.BlockSpec((n // 4, 128), lambda i: (i, 0)),
        out_shape=jax.ShapeDtypeStruct((n, 128), jnp.float32),
    )(a.reshape(n, 128)).reshape(a.shape)


_mesh = plsc.VectorSubcoreMesh(core_axis_name="c", subcore_axis_name="s")


@functools.partial(
    pl.kernel,
    out_type=[
        jax.ShapeDtypeStruct((B * C,), jnp.float32),
        jax.ShapeDtypeStruct((B,), jnp.int32),
        jax.ShapeDtypeStruct((B * D,), jnp.float32),
    ],
    mesh=_mesh,
    compiler_params=pltpu.CompilerParams(needs_layout_passes=False),
    scratch_types=[
        pltpu.VMEM((TPR, 8, D), jnp.float32),
        pltpu.VMEM((C,), jnp.float32),
        pltpu.VMEM((BPW,), jnp.int32),
        pltpu.VMEM((BPW * D,), jnp.float32),
        pltpu.SemaphoreType.DMA,
    ],
)
def _sc_norms(x46_hbm, clssq_hbm, idx_hbm, masked_hbm,
              slab_v, clsrow_v, idx_v, out_v, sem):
    wid = lax.axis_index("s") * NC + lax.axis_index("c")
    base = wid * BPW
    lane = lax.broadcasted_iota(jnp.int32, (L,), 0)

    for g in range(BPW // L):

        def row_body(rr, acc):
            r = g * L + rr
            gr = base + r
            pltpu.sync_copy(x46_hbm.at[pl.ds(gr * TPR, TPR)], slab_v)

            def grp(t2, carry):
                best_s, best_c = carry
                a16 = jnp.zeros((L,), jnp.float32)
                for u in range(L):
                    tidx = 2 * t2 + (u // 8)
                    v0 = slab_v[tidx, u % 8, pl.ds(0, L)]
                    v1 = slab_v[tidx, u % 8, pl.ds(L, L)]
                    v2 = slab_v[tidx, u % 8, pl.ds(2 * L, L)]
                    v3 = slab_v[tidx, u % 8, pl.ds(3 * L, L)]
                    s = jnp.sum(v0 * v0 + v1 * v1 + v2 * v2 + v3 * v3)
                    a16 = jnp.where(lane == u, s, a16)
                    cc = t2 * L + u
                    take = s > best_s
                    best_s = jnp.where(take, s, best_s)
                    best_c = jnp.where(take, cc, best_c)
                clsrow_v[pl.ds(t2 * L, L)] = a16
                return best_s, best_c

            _, cstar = lax.fori_loop(0, C // L, grp,
                                     (jnp.float32(-1.0), jnp.int32(0)))
            pltpu.sync_copy(clsrow_v, clssq_hbm.at[pl.ds(gr * C, C)])
            for m in range(D // L):
                out_v[pl.ds(r * D + m * L, L)] = (
                    slab_v[cstar >> 3, cstar & 7, pl.ds(m * L, L)])
            return jnp.where(lane == rr, cstar, acc)

        acc = lax.fori_loop(0, L, row_body, jnp.zeros((L,), jnp.int32))
        idx_v[pl.ds(g * L, L)] = acc

    pltpu.sync_copy(idx_v, idx_hbm.at[pl.ds(base, BPW)])
    pltpu.sync_copy(out_v, masked_hbm.at[pl.ds(base * D, BPW * D)])


def kernel(x):
    x46 = x.reshape(NTILE, 8, D)
    clssq, idx, masked = _sc_norms(x46)
    cls = _sqrt_tc(clssq)
    return masked.reshape(B, 1, D), idx, cls.reshape(B, C)


# speedup vs baseline: 1.2886x; 1.0018x over previous
import functools

import jax
import jax.numpy as jnp
from jax import lax
from jax.experimental import pallas as pl
from jax.experimental.pallas import tpu as pltpu
from jax.experimental.pallas import tpu_sc as plsc

B, C, D = 1024, 512, 64
NC, NS, L = 2, 16, 16
NW = NC * NS
BPW = B // NW
NTILE = B * C // 8
TPR = C // 8


def _sqrt_body(a_ref, o_ref):
    o_ref[...] = jnp.sqrt(a_ref[...])


def _sqrt_tc(a):
    n = a.shape[0] // 128
    return pl.pallas_call(
        _sqrt_body,
        grid=(4,),
        in_specs=[pl.BlockSpec((n // 4, 128), lambda i: (i, 0))],
        out_specs=pl.BlockSpec((n // 4, 128), lambda i: (i, 0)),
        out_shape=jax.ShapeDtypeStruct((n, 128), jnp.float32),
    )(a.reshape(n, 128)).reshape(a.shape)


_mesh = plsc.VectorSubcoreMesh(core_axis_name="c", subcore_axis_name="s")


@functools.partial(
    pl.kernel,
    out_type=[
        jax.ShapeDtypeStruct((B * C,), jnp.float32),
        jax.ShapeDtypeStruct((B,), jnp.int32),
        jax.ShapeDtypeStruct((B * D,), jnp.float32),
    ],
    mesh=_mesh,
    compiler_params=pltpu.CompilerParams(needs_layout_passes=False),
    scratch_types=[
        pltpu.VMEM((TPR, 8, D), jnp.float32),
        pltpu.VMEM((C,), jnp.float32),
        pltpu.VMEM((BPW,), jnp.int32),
        pltpu.VMEM((BPW * D,), jnp.float32),
        pltpu.SemaphoreType.DMA,
    ],
)
def _sc_norms(x46_hbm, clssq_hbm, idx_hbm, masked_hbm,
              slab_v, clsrow_v, idx_v, out_v, sem):
    wid = lax.axis_index("s") * NC + lax.axis_index("c")
    base = wid * BPW
    lane = lax.broadcasted_iota(jnp.int32, (L,), 0)

    for g in range(BPW // L):

        def row_body(rr, acc):
            r = g * L + rr
            gr = base + r
            pltpu.sync_copy(x46_hbm.at[gr], slab_v)

            def grp(t2, carry):
                best_s, best_c = carry
                a16 = jnp.zeros((L,), jnp.float32)
                for u in range(L):
                    tidx = 2 * t2 + (u // 8)
                    v0 = slab_v[tidx, u % 8, pl.ds(0, L)]
                    v1 = slab_v[tidx, u % 8, pl.ds(L, L)]
                    v2 = slab_v[tidx, u % 8, pl.ds(2 * L, L)]
                    v3 = slab_v[tidx, u % 8, pl.ds(3 * L, L)]
                    s = jnp.sum(v0 * v0 + v1 * v1 + v2 * v2 + v3 * v3)
                    a16 = jnp.where(lane == u, s, a16)
                    cc = t2 * L + u
                    take = s > best_s
                    best_s = jnp.where(take, s, best_s)
                    best_c = jnp.where(take, cc, best_c)
                clsrow_v[pl.ds(t2 * L, L)] = a16
                return best_s, best_c

            _, cstar = lax.fori_loop(0, C // L, grp,
                                     (jnp.float32(-1.0), jnp.int32(0)))
            pltpu.sync_copy(clsrow_v, clssq_hbm.at[pl.ds(gr * C, C)])
            for m in range(D // L):
                out_v[pl.ds(r * D + m * L, L)] = (
                    slab_v[cstar >> 3, cstar & 7, pl.ds(m * L, L)])
            return jnp.where(lane == rr, cstar, acc)

        acc = lax.fori_loop(0, L, row_body, jnp.zeros((L,), jnp.int32))
        idx_v[pl.ds(g * L, L)] = acc

    pltpu.sync_copy(idx_v, idx_hbm.at[pl.ds(base, BPW)])
    pltpu.sync_copy(out_v, masked_hbm.at[pl.ds(base * D, BPW * D)])


def kernel(x):
    x46 = x.reshape(B, TPR, 8, D)
    clssq, idx, masked = _sc_norms(x46)
    cls = _sqrt_tc(clssq)
    return masked.reshape(B, 1, D), idx, cls.reshape(B, C)
